# baseline (speedup 1.0000x reference)
import functools
import math

import jax
import jax.numpy as jnp
from jax import lax
from jax.experimental import pallas as pl
from jax.experimental.pallas import tpu as pltpu
from jax.experimental.pallas import tpu_sc as plsc

D_MODEL = 64
SCALE = math.sqrt(D_MODEL)

NUM_CORES = 2
NUM_SUBCORES = 16
NUM_WORKERS = NUM_CORES * NUM_SUBCORES

CHUNK = 128
LANES = 16


def _emb_kernel(b_per_w, num_chunks, table_hbm, idx_hbm, out_hbm,
                idx_v, rows_v, gsem):
    wid = lax.axis_index("s") * NUM_CORES + lax.axis_index("c")
    base = wid * b_per_w

    pltpu.sync_copy(idx_hbm.at[pl.ds(base, b_per_w)], idx_v)

    def chunk_body(g, _):
        off = g * CHUNK
        pltpu.async_copy(
            table_hbm.at[idx_v.at[pl.ds(off, CHUNK)]],
            rows_v, gsem).wait()

        def row_body(r, _):
            for c in range(D_MODEL // LANES):
                sl = pl.ds(c * LANES, LANES)
                rows_v[r, sl] = rows_v[r, sl] * SCALE
            return 0

        lax.fori_loop(0, CHUNK, row_body, 0)

        pltpu.sync_copy(rows_v, out_hbm.at[pl.ds(base + off, CHUNK)])
        return 0

    lax.fori_loop(0, num_chunks, chunk_body, 0)


def kernel(x, weight):
    batch, hist = x.shape
    vocab, d = weight.shape
    n = batch * hist
    idx = x.reshape(n).astype(jnp.int32)

    b_per_w = n // NUM_WORKERS
    num_chunks = b_per_w // CHUNK

    mesh = plsc.VectorSubcoreMesh(core_axis_name="c", subcore_axis_name="s")
    run = pl.kernel(
        functools.partial(_emb_kernel, b_per_w, num_chunks),
        out_type=jax.ShapeDtypeStruct((n, d), jnp.float32),
        mesh=mesh,
        scratch_types=[
            pltpu.VMEM((b_per_w,), jnp.int32),
            pltpu.VMEM((CHUNK, d), jnp.float32),
            pltpu.SemaphoreType.DMA,
        ],
        compiler_params=pltpu.CompilerParams(use_tc_tiling_on_sc=False),
    )
    out = run(weight, idx)
    return out.reshape(batch, hist, d)


# speedup vs baseline: 1.0753x; 1.0753x over previous
import functools
import math

import jax
import jax.numpy as jnp
from jax import lax
from jax.experimental import pallas as pl
from jax.experimental.pallas import tpu as pltpu
from jax.experimental.pallas import tpu_sc as plsc

D_MODEL = 64
SCALE = math.sqrt(D_MODEL)

NUM_CORES = 2
NUM_SUBCORES = 16
NUM_WORKERS = NUM_CORES * NUM_SUBCORES

CHUNK = 800
LANES = 16


def _scale_chunk(buf):
    @plsc.parallel_loop(0, CHUNK, step=1, unroll=4)
    def _(r):
        for c in range(D_MODEL // LANES):
            sl = pl.ds(c * LANES, LANES)
            buf[r, sl] = buf[r, sl] * SCALE


def _emb_kernel(b_per_w, num_chunks, table_hbm, idx_hbm, out_hbm,
                idx_v, rows_a, rows_b, gsem_a, gsem_b, osem_a, osem_b):
    wid = lax.axis_index("s") * NUM_CORES + lax.axis_index("c")
    base = wid * b_per_w

    pltpu.sync_copy(idx_hbm.at[pl.ds(base, b_per_w)], idx_v)

    def gather_start(g, buf, sem):
        return pltpu.async_copy(
            table_hbm.at[idx_v.at[pl.ds(g * CHUNK, CHUNK)]], buf, sem)

    def gather_wait(buf, sem):
        pltpu.make_async_copy(
            table_hbm.at[idx_v.at[pl.ds(0, CHUNK)]], buf, sem).wait()

    def store_start(g, buf, sem):
        return pltpu.async_copy(buf, out_hbm.at[pl.ds(base + g * CHUNK, CHUNK)], sem)

    def store_wait(buf, sem):
        pltpu.make_async_copy(buf, out_hbm.at[pl.ds(0, CHUNK)], sem).wait()

    n = num_chunks

    gather_start(0, rows_a, gsem_a)
    gather_start(1, rows_b, gsem_b)
    gather_wait(rows_a, gsem_a)
    _scale_chunk(rows_a)
    store_start(0, rows_a, osem_a)

    def pair_body(p, _):
        g1 = 1 + 2 * p
        store_wait(rows_a, osem_a)
        gather_start(g1 + 1, rows_a, gsem_a)
        gather_wait(rows_b, gsem_b)
        _scale_chunk(rows_b)
        store_start(g1, rows_b, osem_b)
        store_wait(rows_b, osem_b)
        gather_start(g1 + 2, rows_b, gsem_b)
        gather_wait(rows_a, gsem_a)
        _scale_chunk(rows_a)
        store_start(g1 + 1, rows_a, osem_a)
        return 0

    lax.fori_loop(0, (n - 2) // 2, pair_body, 0)

    gather_wait(rows_b, gsem_b)
    _scale_chunk(rows_b)
    store_start(n - 1, rows_b, osem_b)
    store_wait(rows_a, osem_a)
    store_wait(rows_b, osem_b)


def kernel(x, weight):
    batch, hist = x.shape
    vocab, d = weight.shape
    n = batch * hist
    idx = x.reshape(n).astype(jnp.int32)

    b_per_w = n // NUM_WORKERS
    num_chunks = b_per_w // CHUNK

    mesh = plsc.VectorSubcoreMesh(core_axis_name="c", subcore_axis_name="s")
    run = pl.kernel(
        functools.partial(_emb_kernel, b_per_w, num_chunks),
        out_type=jax.ShapeDtypeStruct((n, d), jnp.float32),
        mesh=mesh,
        scratch_types=[
            pltpu.VMEM((b_per_w,), jnp.int32),
            pltpu.VMEM((CHUNK, d), jnp.float32),
            pltpu.VMEM((CHUNK, d), jnp.float32),
            pltpu.SemaphoreType.DMA,
            pltpu.SemaphoreType.DMA,
            pltpu.SemaphoreType.DMA,
            pltpu.SemaphoreType.DMA,
        ],
        compiler_params=pltpu.CompilerParams(use_tc_tiling_on_sc=False),
    )
    out = run(weight, idx)
    return out.reshape(batch, hist, d)
